# Initial kernel scaffold; baseline (speedup 1.0000x reference)
import jax
import jax.numpy as jnp
from jax.experimental import pallas as pl


def kernel(in_feat, edge_index, W0, b0, W1, b1, W2, b2):
    raise NotImplementedError("write your pallas kernel here")


# speedup vs baseline: 7.7952x; 7.7952x over previous
import functools

import jax
import jax.numpy as jnp
from jax import lax
from jax.experimental import pallas as pl
from jax.experimental.pallas import tpu as pltpu
from jax.experimental.pallas import tpu_sc as plsc

N = 50000
E = 800000
LANES = 16
N_PAD = 51200
CHUNK = 128
N_WORKERS = 32
EDGES_PER_W = 25088
E_PAD = N_WORKERS * EDGES_PER_W
CHUNKS_PER_W = EDGES_PER_W // CHUNK
ROWS_PER_TILE = N_PAD // 16
ZCHUNKS = ROWS_PER_TILE // CHUNK

_mesh = plsc.VectorSubcoreMesh(core_axis_name="c", subcore_axis_name="s")

_sc_params = pltpu.CompilerParams(use_tc_tiling_on_sc=False)

_f32 = jnp.float32


def _zero_rows(buf, value=0.0):
    @pl.loop(0, CHUNK)
    def _(r):
        buf.at[r][...] = jnp.full((LANES,), value, _f32)


@functools.partial(
    pl.kernel,
    mesh=_mesh,
    out_type=jax.ShapeDtypeStruct((2, N_PAD, LANES), _f32),
    scratch_types=[
        pltpu.VMEM((CHUNK,), jnp.int32),
        pltpu.VMEM((CHUNK,), jnp.int32),
        pltpu.VMEM((CHUNK, LANES), _f32),
        pltpu.VMEM((CHUNK, LANES), _f32),
        pltpu.VMEM_SHARED((N_PAD, LANES), _f32),
    ],
    compiler_params=_sc_params,
)
def _propagate(table_hbm, src_hbm, dst_hbm, out_hbm,
               src_idx, dst_idx, rows, zbuf, acc):
    core = lax.axis_index("c")
    sub = lax.axis_index("s")
    _zero_rows(zbuf)

    @pl.loop(0, ZCHUNKS)
    def _(i):
        pltpu.sync_copy(zbuf, acc.at[pl.ds((sub * ZCHUNKS + i) * CHUNK, CHUNK)])

    plsc.subcore_barrier()
    w = core * 16 + sub

    @pl.loop(0, CHUNKS_PER_W)
    def _(i):
        base = w * EDGES_PER_W + i * CHUNK
        pltpu.sync_copy(src_hbm.at[pl.ds(base, CHUNK)], src_idx)
        pltpu.sync_copy(dst_hbm.at[pl.ds(base, CHUNK)], dst_idx)
        pltpu.sync_copy(table_hbm.at[src_idx], rows)
        pltpu.sync_copy(rows, acc.at[dst_idx], add=True)

    plsc.subcore_barrier()

    @pl.loop(0, ZCHUNKS)
    def _(i):
        off = (sub * ZCHUNKS + i) * CHUNK
        pltpu.sync_copy(acc.at[pl.ds(off, CHUNK)],
                        out_hbm.at[core, pl.ds(off, CHUNK)])


@functools.partial(
    pl.kernel,
    mesh=_mesh,
    out_type=jax.ShapeDtypeStruct((2, N_PAD, LANES), _f32),
    scratch_types=[
        pltpu.VMEM((CHUNK,), jnp.int32),
        pltpu.VMEM((CHUNK, LANES), _f32),
        pltpu.VMEM((CHUNK, LANES), _f32),
        pltpu.VMEM_SHARED((N_PAD, LANES), _f32),
    ],
    compiler_params=_sc_params,
)
def _histogram(idx_hbm, out_hbm, idx, ones, zbuf, acc):
    core = lax.axis_index("c")
    sub = lax.axis_index("s")
    _zero_rows(zbuf)
    _zero_rows(ones, 1.0)

    @pl.loop(0, ZCHUNKS)
    def _(i):
        off = (sub * ZCHUNKS + i) * CHUNK
        pltpu.sync_copy(zbuf, acc.at[pl.ds(off, CHUNK)])

    plsc.subcore_barrier()
    w = core * 16 + sub

    @pl.loop(0, CHUNKS_PER_W)
    def _(i):
        base = w * EDGES_PER_W + i * CHUNK
        pltpu.sync_copy(idx_hbm.at[pl.ds(base, CHUNK)], idx)
        pltpu.sync_copy(ones, acc.at[idx], add=True)

    plsc.subcore_barrier()

    @pl.loop(0, ZCHUNKS)
    def _(i):
        off = (sub * ZCHUNKS + i) * CHUNK
        pltpu.sync_copy(acc.at[pl.ds(off, CHUNK)],
                        out_hbm.at[core, pl.ds(off, CHUNK)])


_TC_R = 2048
_TC_GRID = N_PAD // _TC_R


def _row_spec(width):
    return pl.BlockSpec((_TC_R, width), lambda i: (i, 0))


def _full_spec(shape):
    nd = len(shape)
    return pl.BlockSpec(shape, lambda i, _n=nd: (0,) * _n)


def _tc0_body(dgo0, dgo1, dgi0, dgi1, xf, ns_o, nd_o, t1_o):
    deg_o = dgo0[:, 0] + dgo1[:, 0]
    deg_i = dgi0[:, 0] + dgi1[:, 0]
    ns = 1.0 / jnp.sqrt(jnp.maximum(deg_o, 1.0))
    nd = 1.0 / jnp.sqrt(jnp.maximum(deg_i, 1.0))
    ns_o[...] = ns[:, None]
    nd_o[...] = nd[:, None]
    t1 = xf[:, 0] * ns
    col = lax.broadcasted_iota(jnp.int32, (_TC_R, LANES), 1)
    t1_o[...] = jnp.where(col == 0, t1[:, None], 0.0)


def _tc0(dgo0, dgo1, dgi0, dgi1, xf):
    return pl.pallas_call(
        _tc0_body,
        grid=(_TC_GRID,),
        in_specs=[_row_spec(LANES)] * 4 + [_row_spec(1)],
        out_specs=[_row_spec(1), _row_spec(1), _row_spec(LANES)],
        out_shape=[
            jax.ShapeDtypeStruct((N_PAD, 1), _f32),
            jax.ShapeDtypeStruct((N_PAD, 1), _f32),
            jax.ShapeDtypeStruct((N_PAD, LANES), _f32),
        ],
    )(dgo0, dgo1, dgi0, dgi1, xf)


def _tca_body(p1a, p1b, ns, nd, W0, b0, W1, t2_o):
    a = (p1a[:, 0] + p1b[:, 0]) * nd[:, 0]
    h = a[:, None] * W0[0][None, :] + b0[0][None, :]
    h = jnp.where(h > 0, h, 0.01 * h)
    h = h * ns[...]
    t2_o[...] = jnp.dot(h, W1[...], preferred_element_type=_f32,
                        precision=lax.Precision.HIGHEST)


def _tca(p1a, p1b, ns, nd, W0, b0, W1p):
    return pl.pallas_call(
        _tca_body,
        grid=(_TC_GRID,),
        in_specs=[_row_spec(LANES), _row_spec(LANES), _row_spec(1),
                  _row_spec(1), _full_spec((1, 100)), _full_spec((1, 100)),
                  _full_spec((100, LANES))],
        out_specs=_row_spec(LANES),
        out_shape=jax.ShapeDtypeStruct((N_PAD, LANES), _f32),
    )(p1a, p1b, ns, nd, W0, b0, W1p)


def _tcb_body(p2a, p2b, ns, nd, b1, W2, t3_o):
    p2 = p2a[...] + p2b[...]
    h = p2 * nd[...] + b1[0][None, :]
    h = jnp.maximum(h, 0.0)
    h = h * ns[...]
    t3_o[...] = jnp.dot(h, W2[...], preferred_element_type=_f32,
                        precision=lax.Precision.HIGHEST)


def _tcb(p2a, p2b, ns, nd, b1p, W2p):
    return pl.pallas_call(
        _tcb_body,
        grid=(_TC_GRID,),
        in_specs=[_row_spec(LANES), _row_spec(LANES), _row_spec(1),
                  _row_spec(1), _full_spec((1, LANES)),
                  _full_spec((LANES, LANES))],
        out_specs=_row_spec(LANES),
        out_shape=jax.ShapeDtypeStruct((N_PAD, LANES), _f32),
    )(p2a, p2b, ns, nd, b1p, W2p)


def _tcc_body(p3a, p3b, nd, b2, o):
    p3 = (p3a[:, 0] + p3b[:, 0]) * nd[:, 0] + b2[0, 0]
    o[...] = jnp.maximum(p3, 0.0)[:, None]


def _tcc(p3a, p3b, nd, b2):
    return pl.pallas_call(
        _tcc_body,
        grid=(_TC_GRID,),
        in_specs=[_row_spec(LANES), _row_spec(LANES), _row_spec(1),
                  _full_spec((1, 1))],
        out_specs=_row_spec(1),
        out_shape=jax.ShapeDtypeStruct((N_PAD, 1), _f32),
    )(p3a, p3b, nd, b2)


def kernel(in_feat, edge_index, W0, b0, W1, b1, W2, b2):
    ei = edge_index.astype(jnp.int32)
    pad = jnp.full((E_PAD - E,), N, jnp.int32)
    src = jnp.concatenate([ei[0], pad])
    dst = jnp.concatenate([ei[1], pad])
    xf = jnp.pad(in_feat, ((0, N_PAD - N), (0, 0)))
    W1p = jnp.pad(W1, ((0, 0), (0, LANES - W1.shape[1])))
    W2p = jnp.zeros((LANES, LANES), _f32).at[:W2.shape[0], :W2.shape[1]].set(W2)
    b0r = b0[None, :]
    b1p = jnp.pad(b1, (0, LANES - b1.shape[0]))[None, :]
    b2r = b2[None, :]

    dgo = _histogram(src)
    dgi = _histogram(dst)
    ns, nd, t1 = _tc0(dgo[0], dgo[1], dgi[0], dgi[1], xf)
    p1 = _propagate(t1, src, dst)
    t2 = _tca(p1[0], p1[1], ns, nd, W0, b0r, W1p)
    p2 = _propagate(t2, src, dst)
    t3 = _tcb(p2[0], p2[1], ns, nd, b1p, W2p)
    p3 = _propagate(t3, src, dst)
    out = _tcc(p3[0], p3[1], nd, b2r)
    return out[:N]


# speedup vs baseline: 17.0995x; 2.1936x over previous
import functools

import jax
import jax.numpy as jnp
from jax import lax
from jax.experimental import pallas as pl
from jax.experimental.pallas import tpu as pltpu
from jax.experimental.pallas import tpu_sc as plsc

N = 50000
E = 800000
LANES = 16
N_PAD = 51200
CHUNK = 128
N_WORKERS = 32
NCH = 196
EDGES_PER_W = NCH * CHUNK
E_PAD = N_WORKERS * EDGES_PER_W
S = 4
G = 2
ROWS_PER_TILE = N_PAD // 16
ZCHUNKS = ROWS_PER_TILE // CHUNK

_mesh = plsc.VectorSubcoreMesh(core_axis_name="c", subcore_axis_name="s")
_sc_params = pltpu.CompilerParams(use_tc_tiling_on_sc=False)
_f32 = jnp.float32


def _zero_rows(buf, value=0.0):
    @pl.loop(0, CHUNK)
    def _(r):
        buf.at[r][...] = jnp.full((LANES,), value, _f32)


@functools.partial(
    pl.kernel,
    mesh=_mesh,
    out_type=jax.ShapeDtypeStruct((2, N_PAD, LANES), _f32),
    scratch_types=[
        pltpu.VMEM((NCH, CHUNK), jnp.int32),
        pltpu.VMEM((NCH, CHUNK), jnp.int32),
        pltpu.VMEM((S, CHUNK, LANES), _f32),
        pltpu.VMEM((CHUNK, LANES), _f32),
        pltpu.VMEM_SHARED((N_PAD, LANES), _f32),
        pltpu.SemaphoreType.DMA,
        pltpu.SemaphoreType.DMA((S,)),
        pltpu.SemaphoreType.DMA((S,)),
    ],
    compiler_params=_sc_params,
)
def _propagate(table_hbm, src_hbm, dst_hbm, out_hbm,
               src_all, dst_all, rows, zbuf, acc, isem, gsem, ssem):
    core = lax.axis_index("c")
    sub = lax.axis_index("s")
    w = core * 16 + sub
    cp_s = pltpu.async_copy(src_hbm.at[w], src_all, isem)
    cp_d = pltpu.async_copy(dst_hbm.at[w], dst_all, isem)
    _zero_rows(zbuf)

    @pl.loop(0, ZCHUNKS)
    def _(i):
        pltpu.sync_copy(zbuf, acc.at[pl.ds((sub * ZCHUNKS + i) * CHUNK, CHUNK)])

    cp_s.wait()
    cp_d.wait()
    plsc.subcore_barrier()

    def start_g(j, b):
        pltpu.async_copy(table_hbm.at[src_all.at[j]], rows.at[b], gsem.at[b])

    def wait_g(b, i):
        pltpu.make_async_copy(table_hbm.at[src_all.at[i]], rows.at[b],
                              gsem.at[b]).wait()

    def start_s(i, b):
        pltpu.sync_copy(rows.at[b], acc.at[dst_all.at[i]], add=True)

    def wait_s(b, i):
        pass

    start_g(0, 0)
    start_g(1, 1)
    start_g(2, 2)
    wait_g(0, 0)
    start_s(0, 0)
    start_g(3, 3)
    wait_g(1, 1)
    start_s(1, 1)
    wait_s(0, 0)
    start_g(4, 0)
    wait_g(2, 2)
    start_s(2, 2)
    wait_s(1, 1)
    start_g(5, 1)
    wait_g(3, 3)
    start_s(3, 3)

    @pl.loop(1, NCH // S - 1)
    def _(g):
        i0 = g * S
        for b in range(S):
            i = i0 + b
            j = i + G
            bj = (b + G) % S
            wait_s(bj, i)
            start_g(j, bj)
            wait_g(b, i)
            start_s(i, b)

    i0 = NCH - S
    wait_s(2, 0)
    start_g(i0 + 2, 2)
    wait_g(0, 0)
    start_s(i0 + 0, 0)
    wait_s(3, 0)
    start_g(i0 + 3, 3)
    wait_g(1, 1)
    start_s(i0 + 1, 1)
    wait_g(2, 2)
    start_s(i0 + 2, 2)
    wait_g(3, 3)
    start_s(i0 + 3, 3)
    for b in range(S):
        wait_s(b, 0)
    plsc.subcore_barrier()

    @pl.loop(0, ZCHUNKS)
    def _(i):
        off = (sub * ZCHUNKS + i) * CHUNK
        pltpu.sync_copy(acc.at[pl.ds(off, CHUNK)],
                        out_hbm.at[core, pl.ds(off, CHUNK)])


@functools.partial(
    pl.kernel,
    mesh=_mesh,
    out_type=jax.ShapeDtypeStruct((2, N_PAD, LANES), _f32),
    scratch_types=[
        pltpu.VMEM((NCH, CHUNK), jnp.int32),
        pltpu.VMEM((NCH, CHUNK), jnp.int32),
        pltpu.VMEM((CHUNK, LANES), _f32),
        pltpu.VMEM((CHUNK, LANES), _f32),
        pltpu.VMEM((CHUNK, LANES), _f32),
        pltpu.VMEM_SHARED((N_PAD, LANES), _f32),
        pltpu.SemaphoreType.DMA,
        pltpu.SemaphoreType.DMA((S,)),
    ],
    compiler_params=_sc_params,
)
def _degrees(src_hbm, dst_hbm, out_hbm,
             src_all, dst_all, ones_src, ones_dst, zbuf, acc, isem, ssem):
    core = lax.axis_index("c")
    sub = lax.axis_index("s")
    w = core * 16 + sub
    cp_s = pltpu.async_copy(src_hbm.at[w], src_all, isem)
    cp_d = pltpu.async_copy(dst_hbm.at[w], dst_all, isem)
    _zero_rows(zbuf)
    lane = lax.iota(jnp.int32, LANES)

    @pl.loop(0, CHUNK)
    def _(r):
        ones_src.at[r][...] = jnp.where(lane < 8, 1.0, 0.0).astype(_f32)
        ones_dst.at[r][...] = jnp.where(lane < 8, 0.0, 1.0).astype(_f32)

    @pl.loop(0, ZCHUNKS)
    def _(i):
        pltpu.sync_copy(zbuf, acc.at[pl.ds((sub * ZCHUNKS + i) * CHUNK, CHUNK)])

    cp_s.wait()
    cp_d.wait()
    plsc.subcore_barrier()

    def start2(i, b):
        pltpu.sync_copy(ones_src, acc.at[src_all.at[i]], add=True)
        pltpu.sync_copy(ones_dst, acc.at[dst_all.at[i]], add=True)

    def wait2(b, i):
        pass

    for b in range(S):
        start2(b, b)

    @pl.loop(1, NCH // S)
    def _(g):
        i0 = g * S
        for b in range(S):
            i = i0 + b
            wait2(b, i)
            start2(i, b)

    for b in range(S):
        wait2(b, 0)
    plsc.subcore_barrier()

    @pl.loop(0, ZCHUNKS)
    def _(i):
        off = (sub * ZCHUNKS + i) * CHUNK
        pltpu.sync_copy(acc.at[pl.ds(off, CHUNK)],
                        out_hbm.at[core, pl.ds(off, CHUNK)])


_TC_R = 2048
_TC_GRID = N_PAD // _TC_R


def _row_spec(width):
    return pl.BlockSpec((_TC_R, width), lambda i: (i, 0))


def _full_spec(shape):
    nd = len(shape)
    return pl.BlockSpec(shape, lambda i, _n=nd: (0,) * _n)


def _tc0_body(dg0, dg1, xf, ns_o, nd_o, t1_o):
    deg_o = dg0[:, 0] + dg1[:, 0]
    deg_i = dg0[:, 8] + dg1[:, 8]
    ns = 1.0 / jnp.sqrt(jnp.maximum(deg_o, 1.0))
    nd = 1.0 / jnp.sqrt(jnp.maximum(deg_i, 1.0))
    ns_o[...] = ns[:, None]
    nd_o[...] = nd[:, None]
    t1 = xf[:, 0] * ns
    col = lax.broadcasted_iota(jnp.int32, (_TC_R, LANES), 1)
    t1_o[...] = jnp.where(col == 0, t1[:, None], 0.0)


def _tc0(dg0, dg1, xf):
    return pl.pallas_call(
        _tc0_body,
        grid=(_TC_GRID,),
        in_specs=[_row_spec(LANES)] * 2 + [_row_spec(1)],
        out_specs=[_row_spec(1), _row_spec(1), _row_spec(LANES)],
        out_shape=[
            jax.ShapeDtypeStruct((N_PAD, 1), _f32),
            jax.ShapeDtypeStruct((N_PAD, 1), _f32),
            jax.ShapeDtypeStruct((N_PAD, LANES), _f32),
        ],
    )(dg0, dg1, xf)


def _tca_body(p1a, p1b, ns, nd, W0, b0, W1, t2_o):
    a = (p1a[:, 0] + p1b[:, 0]) * nd[:, 0]
    h = a[:, None] * W0[0][None, :] + b0[0][None, :]
    h = jnp.where(h > 0, h, 0.01 * h)
    h = h * ns[...]
    t2_o[...] = jnp.dot(h, W1[...], preferred_element_type=_f32,
                        precision=lax.Precision.HIGHEST)


def _tca(p1a, p1b, ns, nd, W0, b0, W1p):
    return pl.pallas_call(
        _tca_body,
        grid=(_TC_GRID,),
        in_specs=[_row_spec(LANES), _row_spec(LANES), _row_spec(1),
                  _row_spec(1), _full_spec((1, 100)), _full_spec((1, 100)),
                  _full_spec((100, LANES))],
        out_specs=_row_spec(LANES),
        out_shape=jax.ShapeDtypeStruct((N_PAD, LANES), _f32),
    )(p1a, p1b, ns, nd, W0, b0, W1p)


def _tcb_body(p2a, p2b, ns, nd, b1, W2, t3_o):
    p2 = p2a[...] + p2b[...]
    h = p2 * nd[...] + b1[0][None, :]
    h = jnp.maximum(h, 0.0)
    h = h * ns[...]
    t3_o[...] = jnp.dot(h, W2[...], preferred_element_type=_f32,
                        precision=lax.Precision.HIGHEST)


def _tcb(p2a, p2b, ns, nd, b1p, W2p):
    return pl.pallas_call(
        _tcb_body,
        grid=(_TC_GRID,),
        in_specs=[_row_spec(LANES), _row_spec(LANES), _row_spec(1),
                  _row_spec(1), _full_spec((1, LANES)),
                  _full_spec((LANES, LANES))],
        out_specs=_row_spec(LANES),
        out_shape=jax.ShapeDtypeStruct((N_PAD, LANES), _f32),
    )(p2a, p2b, ns, nd, b1p, W2p)


def _tcc_body(p3a, p3b, nd, b2, o):
    p3 = (p3a[:, 0] + p3b[:, 0]) * nd[:, 0] + b2[0, 0]
    o[...] = jnp.maximum(p3, 0.0)[:, None]


def _tcc(p3a, p3b, nd, b2):
    return pl.pallas_call(
        _tcc_body,
        grid=(_TC_GRID,),
        in_specs=[_row_spec(LANES), _row_spec(LANES), _row_spec(1),
                  _full_spec((1, 1))],
        out_specs=_row_spec(1),
        out_shape=jax.ShapeDtypeStruct((N_PAD, 1), _f32),
    )(p3a, p3b, nd, b2)


def kernel(in_feat, edge_index, W0, b0, W1, b1, W2, b2):
    ei = edge_index.astype(jnp.int32)
    pad = jnp.full((E_PAD - E,), N, jnp.int32)
    src = jnp.concatenate([ei[0], pad]).reshape(N_WORKERS, NCH, CHUNK)
    dst = jnp.concatenate([ei[1], pad]).reshape(N_WORKERS, NCH, CHUNK)
    xf = jnp.pad(in_feat, ((0, N_PAD - N), (0, 0)))
    W1p = jnp.pad(W1, ((0, 0), (0, LANES - W1.shape[1])))
    W2p = jnp.zeros((LANES, LANES), _f32).at[:W2.shape[0], :W2.shape[1]].set(W2)
    b0r = b0[None, :]
    b1p = jnp.pad(b1, (0, LANES - b1.shape[0]))[None, :]
    b2r = b2[None, :]

    dg = _degrees(src, dst)
    ns, nd, t1 = _tc0(dg[0], dg[1], xf)
    p1 = _propagate(t1, src, dst)
    t2 = _tca(p1[0], p1[1], ns, nd, W0, b0r, W1p)
    p2 = _propagate(t2, src, dst)
    t3 = _tcb(p2[0], p2[1], ns, nd, b1p, W2p)
    p3 = _propagate(t3, src, dst)
    out = _tcc(p3[0], p3[1], nd, b2r)
    return out[:N]


# speedup vs baseline: 18.4201x; 1.0772x over previous
import functools

import jax
import jax.numpy as jnp
from jax import lax
from jax.experimental import pallas as pl
from jax.experimental.pallas import tpu as pltpu
from jax.experimental.pallas import tpu_sc as plsc

N = 50000
E = 800000
LANES = 16
N_PAD = 51200
CHUNK = 128
N_WORKERS = 32
NCH = 196
EDGES_PER_W = NCH * CHUNK
E_PAD = N_WORKERS * EDGES_PER_W
S = 4
G = 2
ROWS_PER_TILE = N_PAD // 16
ZCHUNKS = ROWS_PER_TILE // CHUNK

_mesh = plsc.VectorSubcoreMesh(core_axis_name="c", subcore_axis_name="s")
_sc_params = pltpu.CompilerParams(use_tc_tiling_on_sc=False)
_f32 = jnp.float32


def _zero_rows(buf, value=0.0):
    @pl.loop(0, CHUNK)
    def _(r):
        buf.at[r][...] = jnp.full((LANES,), value, _f32)


@functools.partial(
    pl.kernel,
    mesh=_mesh,
    out_type=jax.ShapeDtypeStruct((2, N_PAD, LANES), _f32),
    scratch_types=[
        pltpu.VMEM((NCH, CHUNK), jnp.int32),
        pltpu.VMEM((NCH, CHUNK), jnp.int32),
        pltpu.VMEM((2, S, CHUNK, LANES), _f32),
        pltpu.VMEM((CHUNK, LANES), _f32),
        pltpu.VMEM_SHARED((N_PAD, LANES), _f32),
        pltpu.SemaphoreType.DMA,
        pltpu.SemaphoreType.DMA((2, S)),
        pltpu.SemaphoreType.DMA((S,)),
    ],
    compiler_params=_sc_params,
)
def _propagate(table_hbm, src_hbm, dst_hbm, out_hbm,
               src_all, dst_all, rows, zbuf, acc, isem, gsem, ssem):
    core = lax.axis_index("c")
    sub = lax.axis_index("s")
    w = core * 16 + sub
    cp_s = pltpu.async_copy(src_hbm.at[w], src_all, isem)
    cp_d = pltpu.async_copy(dst_hbm.at[w], dst_all, isem)
    _zero_rows(zbuf)

    @pl.loop(0, ZCHUNKS)
    def _(i):
        pltpu.sync_copy(zbuf, acc.at[pl.ds((sub * ZCHUNKS + i) * CHUNK, CHUNK)])

    cp_s.wait()
    cp_d.wait()
    plsc.subcore_barrier()

    def start_g(j, bank, b):
        pltpu.async_copy(table_hbm.at[src_all.at[j]], rows.at[bank, b],
                         gsem.at[bank, b])

    def wait_g(bank, b, i):
        pltpu.make_async_copy(table_hbm.at[src_all.at[i]],
                              rows.at[bank, b], gsem.at[bank, b]).wait()

    def start_s(i, bank, b):
        return pltpu.async_copy(rows.at[bank, b], acc.at[dst_all.at[i]],
                                ssem.at[b], add=True)

    def cycle(c, bank, nextbank, prefetch):
        base = c * S
        if prefetch:
            for b in range(S):
                start_g(base + S + b, nextbank, b)
        hs = []
        for b in range(S):
            wait_g(bank, b, base + b)
            hs.append(start_s(base + b, bank, b))
        for h in hs:
            h.wait()

    for b in range(S):
        start_g(b, 0, b)

    NCYC = NCH // S

    @pl.loop(0, (NCYC - 1) // 2)
    def _(d):
        c = d * 2
        cycle(c, 0, 1, True)
        cycle(c + 1, 1, 0, True)

    cycle(NCYC - 1, 0, 1, False)
    plsc.subcore_barrier()

    @pl.loop(0, ZCHUNKS)
    def _(i):
        off = (sub * ZCHUNKS + i) * CHUNK
        pltpu.sync_copy(acc.at[pl.ds(off, CHUNK)],
                        out_hbm.at[core, pl.ds(off, CHUNK)])


@functools.partial(
    pl.kernel,
    mesh=_mesh,
    out_type=jax.ShapeDtypeStruct((2, N_PAD, LANES), _f32),
    scratch_types=[
        pltpu.VMEM((NCH, CHUNK), jnp.int32),
        pltpu.VMEM((NCH, CHUNK), jnp.int32),
        pltpu.VMEM((CHUNK, LANES), _f32),
        pltpu.VMEM((CHUNK, LANES), _f32),
        pltpu.VMEM((CHUNK, LANES), _f32),
        pltpu.VMEM_SHARED((N_PAD, LANES), _f32),
        pltpu.SemaphoreType.DMA,
        pltpu.SemaphoreType.DMA((S,)),
    ],
    compiler_params=_sc_params,
)
def _degrees(src_hbm, dst_hbm, out_hbm,
             src_all, dst_all, ones_src, ones_dst, zbuf, acc, isem, ssem):
    core = lax.axis_index("c")
    sub = lax.axis_index("s")
    w = core * 16 + sub
    cp_s = pltpu.async_copy(src_hbm.at[w], src_all, isem)
    cp_d = pltpu.async_copy(dst_hbm.at[w], dst_all, isem)
    _zero_rows(zbuf)
    lane = lax.iota(jnp.int32, LANES)

    @pl.loop(0, CHUNK)
    def _(r):
        ones_src.at[r][...] = jnp.where(lane < 8, 1.0, 0.0).astype(_f32)
        ones_dst.at[r][...] = jnp.where(lane < 8, 0.0, 1.0).astype(_f32)

    @pl.loop(0, ZCHUNKS)
    def _(i):
        pltpu.sync_copy(zbuf, acc.at[pl.ds((sub * ZCHUNKS + i) * CHUNK, CHUNK)])

    cp_s.wait()
    cp_d.wait()
    plsc.subcore_barrier()

    def start2(i, b):
        pltpu.sync_copy(ones_src, acc.at[src_all.at[i]], add=True)
        pltpu.sync_copy(ones_dst, acc.at[dst_all.at[i]], add=True)

    def wait2(b, i):
        pass

    for b in range(S):
        start2(b, b)

    @pl.loop(1, NCH // S)
    def _(g):
        i0 = g * S
        for b in range(S):
            i = i0 + b
            wait2(b, i)
            start2(i, b)

    for b in range(S):
        wait2(b, 0)
    plsc.subcore_barrier()

    @pl.loop(0, ZCHUNKS)
    def _(i):
        off = (sub * ZCHUNKS + i) * CHUNK
        pltpu.sync_copy(acc.at[pl.ds(off, CHUNK)],
                        out_hbm.at[core, pl.ds(off, CHUNK)])


_TC_R = 2048
_TC_GRID = N_PAD // _TC_R


def _row_spec(width):
    return pl.BlockSpec((_TC_R, width), lambda i: (i, 0))


def _full_spec(shape):
    nd = len(shape)
    return pl.BlockSpec(shape, lambda i, _n=nd: (0,) * _n)


def _tc0_body(dg0, dg1, xf, ns_o, nd_o, t1_o):
    deg_o = dg0[:, 0] + dg1[:, 0]
    deg_i = dg0[:, 8] + dg1[:, 8]
    ns = 1.0 / jnp.sqrt(jnp.maximum(deg_o, 1.0))
    nd = 1.0 / jnp.sqrt(jnp.maximum(deg_i, 1.0))
    ns_o[...] = ns[:, None]
    nd_o[...] = nd[:, None]
    t1 = xf[:, 0] * ns
    col = lax.broadcasted_iota(jnp.int32, (_TC_R, LANES), 1)
    t1_o[...] = jnp.where(col == 0, t1[:, None], 0.0)


def _tc0(dg0, dg1, xf):
    return pl.pallas_call(
        _tc0_body,
        grid=(_TC_GRID,),
        in_specs=[_row_spec(LANES)] * 2 + [_row_spec(1)],
        out_specs=[_row_spec(1), _row_spec(1), _row_spec(LANES)],
        out_shape=[
            jax.ShapeDtypeStruct((N_PAD, 1), _f32),
            jax.ShapeDtypeStruct((N_PAD, 1), _f32),
            jax.ShapeDtypeStruct((N_PAD, LANES), _f32),
        ],
    )(dg0, dg1, xf)


def _tca_body(p1a, p1b, ns, nd, W0, b0, W1, t2_o):
    a = (p1a[:, 0] + p1b[:, 0]) * nd[:, 0]
    h = a[:, None] * W0[0][None, :] + b0[0][None, :]
    h = jnp.where(h > 0, h, 0.01 * h)
    h = h * ns[...]
    t2_o[...] = jnp.dot(h, W1[...], preferred_element_type=_f32,
                        precision=lax.Precision.HIGHEST)


def _tca(p1a, p1b, ns, nd, W0, b0, W1p):
    return pl.pallas_call(
        _tca_body,
        grid=(_TC_GRID,),
        in_specs=[_row_spec(LANES), _row_spec(LANES), _row_spec(1),
                  _row_spec(1), _full_spec((1, 100)), _full_spec((1, 100)),
                  _full_spec((100, LANES))],
        out_specs=_row_spec(LANES),
        out_shape=jax.ShapeDtypeStruct((N_PAD, LANES), _f32),
    )(p1a, p1b, ns, nd, W0, b0, W1p)


def _tcb_body(p2a, p2b, ns, nd, b1, W2, t3_o):
    p2 = p2a[...] + p2b[...]
    h = p2 * nd[...] + b1[0][None, :]
    h = jnp.maximum(h, 0.0)
    h = h * ns[...]
    t3_o[...] = jnp.dot(h, W2[...], preferred_element_type=_f32,
                        precision=lax.Precision.HIGHEST)


def _tcb(p2a, p2b, ns, nd, b1p, W2p):
    return pl.pallas_call(
        _tcb_body,
        grid=(_TC_GRID,),
        in_specs=[_row_spec(LANES), _row_spec(LANES), _row_spec(1),
                  _row_spec(1), _full_spec((1, LANES)),
                  _full_spec((LANES, LANES))],
        out_specs=_row_spec(LANES),
        out_shape=jax.ShapeDtypeStruct((N_PAD, LANES), _f32),
    )(p2a, p2b, ns, nd, b1p, W2p)


def _tcc_body(p3a, p3b, nd, b2, o):
    p3 = (p3a[:, 0] + p3b[:, 0]) * nd[:, 0] + b2[0, 0]
    o[...] = jnp.maximum(p3, 0.0)[:, None]


def _tcc(p3a, p3b, nd, b2):
    return pl.pallas_call(
        _tcc_body,
        grid=(_TC_GRID,),
        in_specs=[_row_spec(LANES), _row_spec(LANES), _row_spec(1),
                  _full_spec((1, 1))],
        out_specs=_row_spec(1),
        out_shape=jax.ShapeDtypeStruct((N_PAD, 1), _f32),
    )(p3a, p3b, nd, b2)


def kernel(in_feat, edge_index, W0, b0, W1, b1, W2, b2):
    ei = edge_index.astype(jnp.int32)
    pad = jnp.full((E_PAD - E,), N, jnp.int32)
    src = jnp.concatenate([ei[0], pad]).reshape(N_WORKERS, NCH, CHUNK)
    dst = jnp.concatenate([ei[1], pad]).reshape(N_WORKERS, NCH, CHUNK)
    xf = jnp.pad(in_feat, ((0, N_PAD - N), (0, 0)))
    W1p = jnp.pad(W1, ((0, 0), (0, LANES - W1.shape[1])))
    W2p = jnp.zeros((LANES, LANES), _f32).at[:W2.shape[0], :W2.shape[1]].set(W2)
    b0r = b0[None, :]
    b1p = jnp.pad(b1, (0, LANES - b1.shape[0]))[None, :]
    b2r = b2[None, :]

    dg = _degrees(src, dst)
    ns, nd, t1 = _tc0(dg[0], dg[1], xf)
    p1 = _propagate(t1, src, dst)
    t2 = _tca(p1[0], p1[1], ns, nd, W0, b0r, W1p)
    p2 = _propagate(t2, src, dst)
    t3 = _tcb(p2[0], p2[1], ns, nd, b1p, W2p)
    p3 = _propagate(t3, src, dst)
    out = _tcc(p3[0], p3[1], nd, b2r)
    return out[:N]


# speedup vs baseline: 18.7816x; 1.0196x over previous
import functools

import jax
import jax.numpy as jnp
from jax import lax
from jax.experimental import pallas as pl
from jax.experimental.pallas import tpu as pltpu
from jax.experimental.pallas import tpu_sc as plsc

N = 50000
E = 800000
LANES = 16
N_PAD = 51200
CHUNK = 128
N_WORKERS = 32
NCH = 196
EDGES_PER_W = NCH * CHUNK
E_PAD = N_WORKERS * EDGES_PER_W
S = 4
G = 2
ROWS_PER_TILE = N_PAD // 16
ZCHUNKS = ROWS_PER_TILE // CHUNK

_mesh = plsc.VectorSubcoreMesh(core_axis_name="c", subcore_axis_name="s")
_sc_params = pltpu.CompilerParams(use_tc_tiling_on_sc=False)
_f32 = jnp.float32


def _zero_rows(buf, value=0.0):
    @pl.loop(0, CHUNK)
    def _(r):
        buf.at[r][...] = jnp.full((LANES,), value, _f32)


@functools.partial(
    pl.kernel,
    mesh=_mesh,
    out_type=jax.ShapeDtypeStruct((2, N_PAD, LANES), _f32),
    scratch_types=[
        pltpu.VMEM((NCH, CHUNK), jnp.int32),
        pltpu.VMEM((NCH, CHUNK), jnp.int32),
        pltpu.VMEM((2, S, CHUNK, LANES), _f32),
        pltpu.VMEM((CHUNK, LANES), _f32),
        pltpu.VMEM_SHARED((N_PAD, LANES), _f32),
        pltpu.SemaphoreType.DMA,
        pltpu.SemaphoreType.DMA((2, S)),
        pltpu.SemaphoreType.DMA((S,)),
    ],
    compiler_params=_sc_params,
)
def _propagate(table_hbm, src_hbm, dst_hbm, out_hbm,
               src_all, dst_all, rows, zbuf, acc, isem, gsem, ssem):
    core = lax.axis_index("c")
    sub = lax.axis_index("s")
    w = core * 16 + sub
    cp_s = pltpu.async_copy(src_hbm.at[w], src_all, isem)
    cp_d = pltpu.async_copy(dst_hbm.at[w], dst_all, isem)
    _zero_rows(zbuf)

    @pl.loop(0, ZCHUNKS)
    def _(i):
        pltpu.sync_copy(zbuf, acc.at[pl.ds((sub * ZCHUNKS + i) * CHUNK, CHUNK)])

    cp_s.wait()
    cp_d.wait()
    plsc.subcore_barrier()

    def start_g(j, bank, b):
        pltpu.async_copy(table_hbm.at[src_all.at[j]], rows.at[bank, b],
                         gsem.at[bank, b])

    def wait_g(bank, b, i):
        pltpu.make_async_copy(table_hbm.at[src_all.at[i]],
                              rows.at[bank, b], gsem.at[bank, b]).wait()

    def start_s(i, bank, b):
        return pltpu.async_copy(rows.at[bank, b], acc.at[dst_all.at[i]],
                                ssem.at[b], add=True)

    def cycle(c, bank, nextbank, prefetch):
        base = c * S
        if prefetch:
            for b in range(S):
                start_g(base + S + b, nextbank, b)
        hs = []
        for b in range(S):
            wait_g(bank, b, base + b)
            hs.append(start_s(base + b, bank, b))
        for h in hs:
            h.wait()

    for b in range(S):
        start_g(b, 0, b)

    NCYC = NCH // S

    @pl.loop(0, (NCYC - 1) // 2)
    def _(d):
        c = d * 2
        cycle(c, 0, 1, True)
        cycle(c + 1, 1, 0, True)

    cycle(NCYC - 1, 0, 1, False)
    plsc.subcore_barrier()

    @pl.loop(0, ZCHUNKS)
    def _(i):
        off = (sub * ZCHUNKS + i) * CHUNK
        pltpu.sync_copy(acc.at[pl.ds(off, CHUNK)],
                        out_hbm.at[core, pl.ds(off, CHUNK)])


@functools.partial(
    pl.kernel,
    mesh=_mesh,
    out_type=jax.ShapeDtypeStruct((2, N_PAD, LANES), _f32),
    scratch_types=[
        pltpu.VMEM((NCH, CHUNK), jnp.int32),
        pltpu.VMEM((NCH, CHUNK), jnp.int32),
        pltpu.VMEM((CHUNK, LANES), _f32),
        pltpu.VMEM((CHUNK, LANES), _f32),
        pltpu.VMEM((CHUNK, LANES), _f32),
        pltpu.VMEM_SHARED((N_PAD, LANES), _f32),
        pltpu.SemaphoreType.DMA,
        pltpu.SemaphoreType.DMA((2, S)),
    ],
    compiler_params=_sc_params,
)
def _degrees(src_hbm, dst_hbm, out_hbm,
             src_all, dst_all, ones_src, ones_dst, zbuf, acc, isem, ssem):
    core = lax.axis_index("c")
    sub = lax.axis_index("s")
    w = core * 16 + sub
    cp_s = pltpu.async_copy(src_hbm.at[w], src_all, isem)
    cp_d = pltpu.async_copy(dst_hbm.at[w], dst_all, isem)
    _zero_rows(zbuf)
    lane = lax.iota(jnp.int32, LANES)

    @pl.loop(0, CHUNK)
    def _(r):
        ones_src.at[r][...] = jnp.where(lane < 8, 1.0, 0.0).astype(_f32)
        ones_dst.at[r][...] = jnp.where(lane < 8, 0.0, 1.0).astype(_f32)

    @pl.loop(0, ZCHUNKS)
    def _(i):
        pltpu.sync_copy(zbuf, acc.at[pl.ds((sub * ZCHUNKS + i) * CHUNK, CHUNK)])

    cp_s.wait()
    cp_d.wait()
    plsc.subcore_barrier()

    @pl.loop(0, NCH // S)
    def _(g):
        i0 = g * S
        hs = []
        for b in range(S):
            i = i0 + b
            hs.append(pltpu.async_copy(ones_src, acc.at[src_all.at[i]],
                                       ssem.at[0, b], add=True))
            hs.append(pltpu.async_copy(ones_dst, acc.at[dst_all.at[i]],
                                       ssem.at[1, b], add=True))
        for h in hs:
            h.wait()

    plsc.subcore_barrier()

    @pl.loop(0, ZCHUNKS)
    def _(i):
        off = (sub * ZCHUNKS + i) * CHUNK
        pltpu.sync_copy(acc.at[pl.ds(off, CHUNK)],
                        out_hbm.at[core, pl.ds(off, CHUNK)])


_TC_R = 2048
_TC_GRID = N_PAD // _TC_R


def _row_spec(width):
    return pl.BlockSpec((_TC_R, width), lambda i: (i, 0))


def _full_spec(shape):
    nd = len(shape)
    return pl.BlockSpec(shape, lambda i, _n=nd: (0,) * _n)


def _tc0_body(dg0, dg1, xf, ns_o, nd_o, t1_o):
    deg_o = dg0[:, 0] + dg1[:, 0]
    deg_i = dg0[:, 8] + dg1[:, 8]
    ns = 1.0 / jnp.sqrt(jnp.maximum(deg_o, 1.0))
    nd = 1.0 / jnp.sqrt(jnp.maximum(deg_i, 1.0))
    ns_o[...] = ns[:, None]
    nd_o[...] = nd[:, None]
    t1 = xf[:, 0] * ns
    col = lax.broadcasted_iota(jnp.int32, (_TC_R, LANES), 1)
    t1_o[...] = jnp.where(col == 0, t1[:, None], 0.0)


def _tc0(dg0, dg1, xf):
    return pl.pallas_call(
        _tc0_body,
        grid=(_TC_GRID,),
        in_specs=[_row_spec(LANES)] * 2 + [_row_spec(1)],
        out_specs=[_row_spec(1), _row_spec(1), _row_spec(LANES)],
        out_shape=[
            jax.ShapeDtypeStruct((N_PAD, 1), _f32),
            jax.ShapeDtypeStruct((N_PAD, 1), _f32),
            jax.ShapeDtypeStruct((N_PAD, LANES), _f32),
        ],
    )(dg0, dg1, xf)


def _tca_body(p1a, p1b, ns, nd, W0, b0, W1, t2_o):
    a = (p1a[:, 0] + p1b[:, 0]) * nd[:, 0]
    h = a[:, None] * W0[0][None, :] + b0[0][None, :]
    h = jnp.where(h > 0, h, 0.01 * h)
    h = h * ns[...]
    t2_o[...] = jnp.dot(h, W1[...], preferred_element_type=_f32,
                        precision=lax.Precision.HIGHEST)


def _tca(p1a, p1b, ns, nd, W0, b0, W1p):
    return pl.pallas_call(
        _tca_body,
        grid=(_TC_GRID,),
        in_specs=[_row_spec(LANES), _row_spec(LANES), _row_spec(1),
                  _row_spec(1), _full_spec((1, 100)), _full_spec((1, 100)),
                  _full_spec((100, LANES))],
        out_specs=_row_spec(LANES),
        out_shape=jax.ShapeDtypeStruct((N_PAD, LANES), _f32),
    )(p1a, p1b, ns, nd, W0, b0, W1p)


def _tcb_body(p2a, p2b, ns, nd, b1, W2, t3_o):
    p2 = p2a[...] + p2b[...]
    h = p2 * nd[...] + b1[0][None, :]
    h = jnp.maximum(h, 0.0)
    h = h * ns[...]
    t3_o[...] = jnp.dot(h, W2[...], preferred_element_type=_f32,
                        precision=lax.Precision.HIGHEST)


def _tcb(p2a, p2b, ns, nd, b1p, W2p):
    return pl.pallas_call(
        _tcb_body,
        grid=(_TC_GRID,),
        in_specs=[_row_spec(LANES), _row_spec(LANES), _row_spec(1),
                  _row_spec(1), _full_spec((1, LANES)),
                  _full_spec((LANES, LANES))],
        out_specs=_row_spec(LANES),
        out_shape=jax.ShapeDtypeStruct((N_PAD, LANES), _f32),
    )(p2a, p2b, ns, nd, b1p, W2p)


def _tcc_body(p3a, p3b, nd, b2, o):
    p3 = (p3a[:, 0] + p3b[:, 0]) * nd[:, 0] + b2[0, 0]
    o[...] = jnp.maximum(p3, 0.0)[:, None]


def _tcc(p3a, p3b, nd, b2):
    return pl.pallas_call(
        _tcc_body,
        grid=(_TC_GRID,),
        in_specs=[_row_spec(LANES), _row_spec(LANES), _row_spec(1),
                  _full_spec((1, 1))],
        out_specs=_row_spec(1),
        out_shape=jax.ShapeDtypeStruct((N_PAD, 1), _f32),
    )(p3a, p3b, nd, b2)


def kernel(in_feat, edge_index, W0, b0, W1, b1, W2, b2):
    ei = edge_index.astype(jnp.int32)
    pad = jnp.full((E_PAD - E,), N, jnp.int32)
    src = jnp.concatenate([ei[0], pad]).reshape(N_WORKERS, NCH, CHUNK)
    dst = jnp.concatenate([ei[1], pad]).reshape(N_WORKERS, NCH, CHUNK)
    xf = jnp.pad(in_feat, ((0, N_PAD - N), (0, 0)))
    W1p = jnp.pad(W1, ((0, 0), (0, LANES - W1.shape[1])))
    W2p = jnp.zeros((LANES, LANES), _f32).at[:W2.shape[0], :W2.shape[1]].set(W2)
    b0r = b0[None, :]
    b1p = jnp.pad(b1, (0, LANES - b1.shape[0]))[None, :]
    b2r = b2[None, :]

    dg = _degrees(src, dst)
    ns, nd, t1 = _tc0(dg[0], dg[1], xf)
    p1 = _propagate(t1, src, dst)
    t2 = _tca(p1[0], p1[1], ns, nd, W0, b0r, W1p)
    p2 = _propagate(t2, src, dst)
    t3 = _tcb(p2[0], p2[1], ns, nd, b1p, W2p)
    p3 = _propagate(t3, src, dst)
    out = _tcc(p3[0], p3[1], nd, b2r)
    return out[:N]
